# baseline (speedup 1.0000x reference)
import functools

import jax
import jax.numpy as jnp
from jax import lax
from jax.experimental import pallas as pl
from jax.experimental.pallas import tpu as pltpu
from jax.experimental.pallas import tpu_sc as plsc

N = 10000
E = 320000
D = 128
DA = 144

NC = 2
NS = 16
NW = NC * NS

CHUNK = 128
KPT = 80
NCHUNKS = NW * KPT
E_PAD = NCHUNKS * CHUNK

PH = 16

ROWS_T = 632
LAST_T = N - 15 * ROWS_T


def _sc_body(feat_hbm, src_hbm, dst_hbm, sums_out,
             sums_sh, srcall, dstall, rows0, rows1,
             sem_g0, sem_g1, sem_s0, sem_s1):
    cid = lax.axis_index("c")
    sid = lax.axis_index("s")
    wid = sid * NC + cid

    zeros16 = jnp.zeros((16,), jnp.float32)

    def zr_row(r, _):
        def zr_col(j, _):
            rows0[r, pl.ds(pl.multiple_of(j * 16, 16), 16)] = zeros16
            return 0
        return lax.fori_loop(0, DA // 16, zr_col, 0)

    lax.fori_loop(0, CHUNK, zr_row, 0)

    row0 = sid * ROWS_T
    for b in range(4):
        pltpu.sync_copy(rows0, sums_sh.at[pl.ds(row0 + b * CHUNK, CHUNK)])

    @pl.when(sid < NS - 1)
    def _():
        pltpu.sync_copy(rows0.at[pl.ds(0, 120)],
                        sums_sh.at[pl.ds(row0 + 4 * CHUNK, 120)])

    @pl.when(sid == NS - 1)
    def _():
        pltpu.sync_copy(rows0.at[pl.ds(0, 8)],
                        sums_sh.at[pl.ds(row0 + 4 * CHUNK, 8)])

    plsc.subcore_barrier()

    chunk0 = wid * KPT

    def idx_at(ref, j):
        return ref.at[pl.ds(pl.multiple_of(j * CHUNK, CHUNK), CHUNK)]

    def g_start(j, rows, sem):
        pltpu.async_copy(feat_hbm.at[idx_at(srcall, j)], rows, sem)

    def g_wait(j, rows, sem):
        pltpu.make_async_copy(feat_hbm.at[idx_at(srcall, j)], rows,
                              sem).wait()

    def s_start(j, rows, sem):
        pltpu.async_copy(rows, sums_sh.at[idx_at(dstall, j)], sem, add=True)

    def s_wait(j, rows, sem):
        pltpu.make_async_copy(rows, sums_sh.at[idx_at(dstall, j)],
                              sem).wait()

    def run_phase(pbase):
        ebase = pl.multiple_of(pbase * CHUNK, CHUNK)
        pltpu.sync_copy(src_hbm.at[pl.ds(ebase, PH * CHUNK)], srcall)
        pltpu.sync_copy(dst_hbm.at[pl.ds(ebase, PH * CHUNK)], dstall)

        g_start(0, rows0, sem_g0)

        def pair(i, _):
            j0 = 2 * i
            g_wait(j0, rows0, sem_g0)
            s_start(j0, rows0, sem_s0)

            @pl.when(i > 0)
            def _():
                s_wait(j0 - 1, rows1, sem_s1)

            g_start(j0 + 1, rows1, sem_g1)
            g_wait(j0 + 1, rows1, sem_g1)
            s_start(j0 + 1, rows1, sem_s1)
            s_wait(j0, rows0, sem_s0)

            @pl.when(j0 + 2 < PH)
            def _():
                g_start(j0 + 2, rows0, sem_g0)

            return 0

        lax.fori_loop(0, PH // 2, pair, 0)
        s_wait(PH - 1, rows1, sem_s1)

    for p in range(KPT // PH):
        run_phase(chunk0 + p * PH)

    plsc.subcore_barrier()

    out0 = cid * N + row0
    for b in range(4):
        buf = rows0 if b % 2 == 0 else rows1
        pltpu.sync_copy(sums_sh.at[pl.ds(row0 + b * CHUNK, CHUNK)], buf)
        pltpu.sync_copy(buf, sums_out.at[pl.ds(out0 + b * CHUNK, CHUNK)])

    @pl.when(sid < NS - 1)
    def _():
        pltpu.sync_copy(sums_sh.at[pl.ds(row0 + 4 * CHUNK, 120)],
                        rows0.at[pl.ds(0, 120)])
        pltpu.sync_copy(rows0.at[pl.ds(0, 120)],
                        sums_out.at[pl.ds(out0 + 4 * CHUNK, 120)])

    @pl.when(sid == NS - 1)
    def _():
        pltpu.sync_copy(sums_sh.at[pl.ds(row0 + 4 * CHUNK, 8)],
                        rows0.at[pl.ds(0, 8)])
        pltpu.sync_copy(rows0.at[pl.ds(0, 8)],
                        sums_out.at[pl.ds(out0 + 4 * CHUNK, 8)])


_sc_scatter = functools.partial(
    pl.kernel,
    out_type=jax.ShapeDtypeStruct((NC * N, DA), jnp.float32),
    mesh=plsc.VectorSubcoreMesh(core_axis_name="c", subcore_axis_name="s"),
    compiler_params=pltpu.CompilerParams(use_tc_tiling_on_sc=False),
    scratch_types=(
        pltpu.VMEM_SHARED((N, DA), jnp.float32),
        pltpu.VMEM((PH * CHUNK,), jnp.int32),
        pltpu.VMEM((PH * CHUNK,), jnp.int32),
        pltpu.VMEM((CHUNK, DA), jnp.float32),
        pltpu.VMEM((CHUNK, DA), jnp.float32),
        pltpu.SemaphoreType.DMA,
        pltpu.SemaphoreType.DMA,
        pltpu.SemaphoreType.DMA,
        pltpu.SemaphoreType.DMA,
    ),
)(_sc_body)


BLK = 1000


def _tc_body(f_ref, w_ref, s0_ref, s1_ref, o_ref):
    w = w_ref[...]
    s = s0_ref[...] + s1_ref[...]
    cnt = s[:, D:D + 1]
    mean = s[:, :D] * (1.0 / jnp.maximum(cnt, 1.0))
    nodes = jnp.dot(f_ref[...], w, preferred_element_type=jnp.float32)
    agg = jnp.dot(mean, w, preferred_element_type=jnp.float32)
    o_ref[:, :D] = jnp.maximum(nodes, 0.0)
    o_ref[:, D:] = jnp.maximum(agg, 0.0)


def _tc_dense(features, weight, sums2):
    return pl.pallas_call(
        _tc_body,
        grid=(N // BLK,),
        in_specs=[
            pl.BlockSpec((BLK, D), lambda i: (i, 0)),
            pl.BlockSpec((D, D), lambda i: (0, 0)),
            pl.BlockSpec((BLK, DA), lambda i: (i, 0)),
            pl.BlockSpec((BLK, DA), lambda i: (N // BLK + i, 0)),
        ],
        out_specs=pl.BlockSpec((BLK, 2 * D), lambda i: (i, 0)),
        out_shape=jax.ShapeDtypeStruct((N, 2 * D), jnp.float32),
    )(features, weight, sums2, sums2)


def kernel(features, edges, weight):
    edges = edges.astype(jnp.int32)
    npad = E_PAD - E
    spread = jnp.arange(npad, dtype=jnp.int32) % 128
    dst = jnp.concatenate([edges[0], spread])
    src = jnp.concatenate([edges[1], N + spread])
    taug = jnp.concatenate(
        [
            jnp.concatenate(
                [features,
                 jnp.ones((N, 1), jnp.float32),
                 jnp.zeros((N, DA - D - 1), jnp.float32)], axis=1),
            jnp.zeros((128, DA), jnp.float32),
        ],
        axis=0,
    )
    sums2 = _sc_scatter(taug, src, dst)
    return _tc_dense(features, weight, sums2)


# speedup vs baseline: 1.2487x; 1.2487x over previous
import functools

import jax
import jax.numpy as jnp
from jax import lax
from jax.experimental import pallas as pl
from jax.experimental.pallas import tpu as pltpu
from jax.experimental.pallas import tpu_sc as plsc

N = 10000
E = 320000
D = 128

NC = 2
NS = 16
NW = NC * NS

CHUNK = 128
KPT = 80
NCHUNKS = NW * KPT
E_PAD = NCHUNKS * CHUNK
NPAD = N + 128

PH = 40

ROWS_T = 632
TAIL = NPAD - 15 * ROWS_T
DUMP_T = 632
DUMP_LAST = N - 15 * DUMP_T


def _sc_body(feat_hbm, src_hbm, dst_hbm, sums_out, cnts_out,
             sums_sh, cnts_sh, srcall, dstall, rows0, rows1, onesv, zcnt,
             sem_g0, sem_g1, sem_s0, sem_s1, sem_c):
    cid = lax.axis_index("c")
    sid = lax.axis_index("s")
    wid = sid * NC + cid

    zeros16 = jnp.zeros((16,), jnp.float32)
    ones16 = jnp.ones((16,), jnp.float32)

    def zr_row(r, _):
        def zr_col(j, _):
            rows0[r, pl.ds(pl.multiple_of(j * 16, 16), 16)] = zeros16
            return 0
        return lax.fori_loop(0, D // 16, zr_col, 0)

    lax.fori_loop(0, CHUNK, zr_row, 0)

    def zc(i, _):
        zcnt[pl.ds(pl.multiple_of(i * 16, 16), 16)] = zeros16
        return 0

    lax.fori_loop(0, TAIL // 16, zc, 0)

    for j in range(CHUNK // 16):
        onesv[pl.ds(j * 16, 16)] = ones16

    row0 = sid * ROWS_T
    zd = []
    for b in range(4):
        sem = sem_s0 if b % 2 == 0 else sem_s1
        zd.append(pltpu.async_copy(
            rows0, sums_sh.at[pl.ds(row0 + b * CHUNK, CHUNK)], sem))
    zd.append(pltpu.async_copy(rows0.at[pl.ds(0, 120)],
                               sums_sh.at[pl.ds(row0 + 4 * CHUNK, 120)],
                               sem_g0))
    zd.append(pltpu.async_copy(zcnt.at[pl.ds(0, ROWS_T)],
                               cnts_sh.at[pl.ds(row0, ROWS_T)], sem_g1))

    @pl.when(sid == NS - 1)
    def _():
        pltpu.sync_copy(rows0.at[pl.ds(0, 16)],
                        sums_sh.at[pl.ds(NPAD - 16, 16)])
        pltpu.sync_copy(zcnt.at[pl.ds(0, 16)],
                        cnts_sh.at[pl.ds(NPAD - 16, 16)])

    for d in zd:
        d.wait()

    plsc.subcore_barrier()

    chunk0 = wid * KPT

    def idx_at(ref, j):
        return ref.at[pl.ds(pl.multiple_of(j * CHUNK, CHUNK), CHUNK)]

    def g_start(j, rows, sem):
        pltpu.async_copy(feat_hbm.at[idx_at(srcall, j)], rows, sem)

    def g_wait(j, rows, sem):
        pltpu.make_async_copy(feat_hbm.at[idx_at(srcall, j)], rows,
                              sem).wait()

    def s_start(j, rows, sem):
        pltpu.async_copy(rows, sums_sh.at[idx_at(dstall, j)], sem, add=True)
        pltpu.async_copy(onesv, cnts_sh.at[idx_at(dstall, j)], sem_c,
                         add=True)

    def s_wait(j, rows, sem):
        pltpu.make_async_copy(rows, sums_sh.at[idx_at(dstall, j)],
                              sem).wait()

    def run_phase(pbase):
        ebase = pl.multiple_of(pbase * CHUNK, CHUNK)
        pltpu.sync_copy(src_hbm.at[pl.ds(ebase, PH * CHUNK)], srcall)
        pltpu.sync_copy(dst_hbm.at[pl.ds(ebase, PH * CHUNK)], dstall)

        g_start(0, rows0, sem_g0)

        def pair(i, _):
            j0 = 2 * i
            g_wait(j0, rows0, sem_g0)
            s_start(j0, rows0, sem_s0)

            @pl.when(i > 0)
            def _():
                s_wait(j0 - 1, rows1, sem_s1)

            g_start(j0 + 1, rows1, sem_g1)
            g_wait(j0 + 1, rows1, sem_g1)
            s_start(j0 + 1, rows1, sem_s1)
            s_wait(j0, rows0, sem_s0)

            @pl.when(j0 + 2 < PH)
            def _():
                g_start(j0 + 2, rows0, sem_g0)

            return 0

        lax.fori_loop(0, PH // 2, pair, 0)
        s_wait(PH - 1, rows1, sem_s1)

        def cdrain(j, _):
            pltpu.make_async_copy(onesv, cnts_sh.at[idx_at(dstall, j)],
                                  sem_c).wait()
            return 0

        lax.fori_loop(0, PH, cdrain, 0)

    run_phase(chunk0)
    run_phase(chunk0 + PH)

    plsc.subcore_barrier()

    out0 = cid * N + row0
    wr = {}
    for b in range(4):
        buf, sem = (rows0, sem_s0) if b % 2 == 0 else (rows1, sem_s1)
        if b >= 2:
            wr[b - 2].wait()
        pltpu.sync_copy(sums_sh.at[pl.ds(row0 + b * CHUNK, CHUNK)], buf)
        wr[b] = pltpu.async_copy(
            buf, sums_out.at[pl.ds(out0 + b * CHUNK, CHUNK)], sem)
    wr[2].wait()

    @pl.when(sid < NS - 1)
    def _():
        pltpu.sync_copy(sums_sh.at[pl.ds(row0 + 4 * CHUNK, 120)],
                        rows0.at[pl.ds(0, 120)])
        d1 = pltpu.async_copy(rows0.at[pl.ds(0, 120)],
                              sums_out.at[pl.ds(out0 + 4 * CHUNK, 120)],
                              sem_g0)
        pltpu.sync_copy(cnts_sh.at[pl.ds(row0, DUMP_T)],
                        zcnt.at[pl.ds(0, DUMP_T)])
        d2 = pltpu.async_copy(zcnt.at[pl.ds(0, DUMP_T)],
                              cnts_out.at[pl.ds(out0, DUMP_T)], sem_g1)
        d1.wait()
        d2.wait()

    @pl.when(sid == NS - 1)
    def _():
        pltpu.sync_copy(sums_sh.at[pl.ds(row0 + 4 * CHUNK, 8)],
                        rows0.at[pl.ds(0, 8)])
        d1 = pltpu.async_copy(rows0.at[pl.ds(0, 8)],
                              sums_out.at[pl.ds(out0 + 4 * CHUNK, 8)],
                              sem_g0)
        pltpu.sync_copy(cnts_sh.at[pl.ds(row0, DUMP_LAST)],
                        zcnt.at[pl.ds(0, DUMP_LAST)])
        d2 = pltpu.async_copy(zcnt.at[pl.ds(0, DUMP_LAST)],
                              cnts_out.at[pl.ds(out0, DUMP_LAST)], sem_g1)
        d1.wait()
        d2.wait()

    wr[3].wait()


_sc_scatter = functools.partial(
    pl.kernel,
    out_type=(
        jax.ShapeDtypeStruct((NC * N, D), jnp.float32),
        jax.ShapeDtypeStruct((NC * N,), jnp.float32),
    ),
    mesh=plsc.VectorSubcoreMesh(core_axis_name="c", subcore_axis_name="s"),
    scratch_types=(
        pltpu.VMEM_SHARED((NPAD, D), jnp.float32),
        pltpu.VMEM_SHARED((NPAD,), jnp.float32),
        pltpu.VMEM((PH * CHUNK,), jnp.int32),
        pltpu.VMEM((PH * CHUNK,), jnp.int32),
        pltpu.VMEM((CHUNK, D), jnp.float32),
        pltpu.VMEM((CHUNK, D), jnp.float32),
        pltpu.VMEM((CHUNK,), jnp.float32),
        pltpu.VMEM((TAIL,), jnp.float32),
        pltpu.SemaphoreType.DMA,
        pltpu.SemaphoreType.DMA,
        pltpu.SemaphoreType.DMA,
        pltpu.SemaphoreType.DMA,
        pltpu.SemaphoreType.DMA,
    ),
)(_sc_body)


BLK = 1000


def _tc_body(f_ref, w_ref, s0_ref, s1_ref, c0_ref, c1_ref, o_ref):
    w = w_ref[...]
    s = s0_ref[...] + s1_ref[...]
    cnt = c0_ref[...] + c1_ref[...]
    mean = s * (1.0 / jnp.maximum(cnt, 1.0))
    nodes = jnp.dot(f_ref[...], w, preferred_element_type=jnp.float32)
    agg = jnp.dot(mean, w, preferred_element_type=jnp.float32)
    o_ref[:, :D] = jnp.maximum(nodes, 0.0)
    o_ref[:, D:] = jnp.maximum(agg, 0.0)


def _tc_dense(features, weight, sums2, cnts2):
    return pl.pallas_call(
        _tc_body,
        grid=(N // BLK,),
        in_specs=[
            pl.BlockSpec((BLK, D), lambda i: (i, 0)),
            pl.BlockSpec((D, D), lambda i: (0, 0)),
            pl.BlockSpec((BLK, D), lambda i: (i, 0)),
            pl.BlockSpec((BLK, D), lambda i: (N // BLK + i, 0)),
            pl.BlockSpec((BLK, 1), lambda i: (i, 0)),
            pl.BlockSpec((BLK, 1), lambda i: (N // BLK + i, 0)),
        ],
        out_specs=pl.BlockSpec((BLK, 2 * D), lambda i: (i, 0)),
        out_shape=jax.ShapeDtypeStruct((N, 2 * D), jnp.float32),
    )(features, weight, sums2, sums2, cnts2, cnts2)


def kernel(features, edges, weight):
    edges = edges.astype(jnp.int32)
    npad = E_PAD - E
    spread = jnp.arange(npad, dtype=jnp.int32) % 128
    dst = jnp.concatenate([edges[0], N + spread])
    src = jnp.concatenate([edges[1], spread])
    sums2, cnts2 = _sc_scatter(features, src, dst)
    return _tc_dense(features, weight, sums2, cnts2.reshape(NC * N, 1))
